# Initial kernel scaffold; baseline (speedup 1.0000x reference)
import jax
import jax.numpy as jnp
from jax.experimental import pallas as pl


def kernel(equalized_symbol, csi_context, noise_power, rate_one_hot, params):
    raise NotImplementedError("write your pallas kernel here")


# speedup vs baseline: 1.0840x; 1.0840x over previous
import jax
import jax.numpy as jnp
from jax.experimental import pallas as pl
from jax.experimental.pallas import tpu as pltpu

_LATENTS = (32, 64, 96, 128, 192, 256)
_NH = 6
_DIN = 73
_DP = 128
_DMAX = 256
_TILE = 512
_BATCH = 16384


def _fused_body(x_ref, r_ref, w1_ref, b1_ref, a_ref, w2_ref, b2_ref,
                w3_ref, b3_ref, lnw_ref, lnb_ref, o_ref):
    x = x_ref[...]
    best = r_ref[:, 0:1]
    e = jnp.zeros((x.shape[0], 1), jnp.int32)
    for j in range(1, _NH):
        rj = r_ref[:, j:j + 1]
        m = rj > best
        e = jnp.where(m, j, e)
        best = jnp.maximum(best, rj)

    acc = jnp.zeros((x.shape[0], _DMAX), jnp.float32)
    for i in range(_NH):
        d = _LATENTS[i]
        h = jax.lax.dot_general(x, w1_ref[i], (((1,), (1,)), ((), ())),
                                preferred_element_type=jnp.float32)
        h = h + b1_ref[i]
        a1 = a_ref[i, 0]
        h = jnp.where(h >= 0, h, a1 * h)
        h = jax.lax.dot_general(h, w2_ref[i], (((1,), (1,)), ((), ())),
                                preferred_element_type=jnp.float32)
        h = h + b2_ref[i]
        a2 = a_ref[i, 1]
        h = jnp.where(h >= 0, h, a2 * h)
        h = jax.lax.dot_general(h, w3_ref[i], (((1,), (1,)), ((), ())),
                                preferred_element_type=jnp.float32)
        h = h + b3_ref[i]
        mu = jnp.sum(h, axis=1, keepdims=True) * (1.0 / d)
        col = jax.lax.broadcasted_iota(jnp.int32, h.shape, 1)
        diff = jnp.where(col < d, h - mu, 0.0)
        var = jnp.sum(diff * diff, axis=1, keepdims=True) * (1.0 / d)
        y = diff * jax.lax.rsqrt(var + 1e-5) * lnw_ref[i] + lnb_ref[i]
        mask = (e == i).astype(jnp.float32)
        acc = acc + mask * y
    o_ref[...] = acc


def kernel(equalized_symbol, csi_context, noise_power, rate_one_hot, params):
    b = equalized_symbol.shape[0]
    combined = jnp.concatenate(
        [equalized_symbol, csi_context, noise_power[:, None],
         jnp.zeros((b, _DP - _DIN), jnp.float32)], axis=1)

    w1s = jnp.stack([jnp.pad(p['W1'], ((0, 0), (0, _DP - _DIN))) for p in params])
    b1s = jnp.stack([p['b1'][None, :] for p in params])
    w2s = jnp.stack([p['W2'] for p in params])
    b2s = jnp.stack([p['b2'][None, :] for p in params])
    w3s = jnp.stack([jnp.pad(p['W3'], ((0, _DMAX - p['W3'].shape[0]), (0, 0)))
                     for p in params])
    b3s = jnp.stack([jnp.pad(p['b3'], (0, _DMAX - p['b3'].shape[0]))[None, :]
                     for p in params])
    lnws = jnp.stack([jnp.pad(p['ln_w'], (0, _DMAX - p['ln_w'].shape[0]))[None, :]
                      for p in params])
    lnbs = jnp.stack([jnp.pad(p['ln_b'], (0, _DMAX - p['ln_b'].shape[0]))[None, :]
                      for p in params])
    a_all = jnp.stack([jnp.concatenate([p['a1'], p['a2']]) for p in params])

    grid = (b // _TILE,)
    full = lambda shp: pl.BlockSpec(shp, lambda i: (0,) * len(shp))
    out = pl.pallas_call(
        _fused_body,
        grid=grid,
        in_specs=[
            pl.BlockSpec((_TILE, _DP), lambda i: (i, 0)),
            pl.BlockSpec((_TILE, _NH), lambda i: (i, 0)),
            full((_NH, _DP, _DP)),
            full((_NH, 1, _DP)),
            pl.BlockSpec(memory_space=pltpu.SMEM),
            full((_NH, 64, _DP)),
            full((_NH, 1, 64)),
            full((_NH, _DMAX, 64)),
            full((_NH, 1, _DMAX)),
            full((_NH, 1, _DMAX)),
            full((_NH, 1, _DMAX)),
        ],
        out_specs=pl.BlockSpec((_TILE, _DMAX), lambda i: (i, 0)),
        out_shape=jax.ShapeDtypeStruct((b, _DMAX), jnp.float32),
    )(combined, rate_one_hot, w1s, b1s, a_all, w2s, b2s, w3s, b3s, lnws, lnbs)
    return out
